# baseline (speedup 1.0000x reference)
import jax
import jax.numpy as jnp
from jax.experimental import pallas as pl
from jax.experimental.pallas import tpu as pltpu

_TOT = 16_000_000
_BLOCK = 640_000


def _copy_body(src_ref, dst_ref):
    dst_ref[...] = src_ref[...]


def kernel(med_embeddings):
    n, d = med_embeddings.shape
    flat = med_embeddings.reshape(n * d)
    out = pl.pallas_call(
        _copy_body,
        grid=(_TOT // _BLOCK,),
        in_specs=[pl.BlockSpec((_BLOCK,), lambda i: (i,))],
        out_specs=pl.BlockSpec((_BLOCK,), lambda i: (i,)),
        out_shape=jax.ShapeDtypeStruct(flat.shape, flat.dtype),
    )(flat)
    return out.reshape(n, d)


# speedup vs baseline: 1.0591x; 1.0591x over previous
import jax
import jax.numpy as jnp
from jax import lax
from jax.experimental import pallas as pl
from jax.experimental.pallas import tpu as pltpu
from jax.experimental.pallas import tpu_sc as plsc

_ROWS = 1_000_000
_D = 16
_NW = 32
_CH = 496
_NCHT = 2016
_PERW = _NCHT // _NW
_TAIL_OFF = _NCHT * _CH
_TAIL = _ROWS - _TAIL_OFF


def _run(src, dst, buf0, buf1, s_in0, s_in1, s_out0, s_out1):
    wid = lax.axis_index("s") * 2 + lax.axis_index("c")
    bufs = (buf0, buf1)
    sin = (s_in0, s_in1)
    sout = (s_out0, s_out1)

    def mk(g):
        cid = wid * _PERW + g
        off = pl.multiple_of(cid * _CH, 8)
        b = g % 2
        inc = pltpu.make_async_copy(src.at[pl.ds(off, _CH), :], bufs[b], sin[b])
        outc = pltpu.make_async_copy(bufs[b], dst.at[pl.ds(off, _CH), :], sout[b])
        return inc, outc

    cps = [mk(g) for g in range(_PERW)]
    for g in range(_PERW):
        if g >= 2:
            cps[g - 2][1].wait()
        cps[g][0].start()
        if g >= 1:
            cps[g - 1][0].wait()
            cps[g - 1][1].start()
    cps[_PERW - 1][0].wait()
    cps[_PERW - 1][1].start()
    cps[_PERW - 2][1].wait()
    cps[_PERW - 1][1].wait()

    tail_in = pltpu.make_async_copy(
        src.at[pl.ds(_TAIL_OFF, _TAIL), :], buf0.at[pl.ds(0, _TAIL), :], sin[0]
    )
    tail_out = pltpu.make_async_copy(
        buf0.at[pl.ds(0, _TAIL), :], dst.at[pl.ds(_TAIL_OFF, _TAIL), :], sout[0]
    )

    @pl.when(wid == 0)
    def _():
        tail_in.start()
        tail_in.wait()
        tail_out.start()
        tail_out.wait()


def kernel(med_embeddings):
    run = pl.kernel(
        _run,
        out_type=jax.ShapeDtypeStruct((_ROWS, _D), jnp.float32),
        mesh=plsc.VectorSubcoreMesh(core_axis_name="c", subcore_axis_name="s"),
        scratch_types=[
            pltpu.VMEM((_CH, _D), jnp.float32),
            pltpu.VMEM((_CH, _D), jnp.float32),
            pltpu.SemaphoreType.DMA,
            pltpu.SemaphoreType.DMA,
            pltpu.SemaphoreType.DMA,
            pltpu.SemaphoreType.DMA,
        ],
        compiler_params=pltpu.CompilerParams(skip_device_barrier=True),
    )
    return run(med_embeddings)
